# Initial kernel scaffold; baseline (speedup 1.0000x reference)
import jax
import jax.numpy as jnp
from jax.experimental import pallas as pl


def kernel(x, edge_index, batch, W_rel1, b1, W_root1, W_rel2, b2, W_root2, W_loc, b_loc, W_ls, b_ls):
    raise NotImplementedError("write your pallas kernel here")


# speedup vs baseline: 5.0259x; 5.0259x over previous
import jax
import jax.numpy as jnp
from jax import lax
from jax.experimental import pallas as pl
from jax.experimental.pallas import tpu as pltpu
from jax.experimental.pallas import tpu_sc as plsc

N = 16000
E = 256000
IN = 128
HID = 256
LAT = 64
BATCH = 16
N_PER = 1000

NUM_TILES = 16
EDGE_CHUNK = 128
CHUNKS_PER_TILE = E // (NUM_TILES * EDGE_CHUNK)
NODES_PER_TILE = N // NUM_TILES
FLUSH_CHUNK = 125
N_FLUSH = NODES_PER_TILE // FLUSH_CHUNK


def _seg_scratch():
    return [
        pltpu.VMEM((CHUNKS_PER_TILE, EDGE_CHUNK), jnp.int32),
        pltpu.VMEM((CHUNKS_PER_TILE, EDGE_CHUNK), jnp.int32),
        pltpu.VMEM((EDGE_CHUNK, 64), jnp.float32),
        pltpu.VMEM((EDGE_CHUNK, 64), jnp.float32),
        pltpu.VMEM((FLUSH_CHUNK, 64), jnp.float32),
        pltpu.VMEM_SHARED((N, 64), jnp.float32),
        pltpu.SemaphoreType.DMA,
    ]


def _seg_pass(c, s, tbl_pair, out_pair, zeros_h,
              idx_s, idx_d, rowsA, rowsB, zbuf, accum, gsem):
    nbase = s * NODES_PER_TILE
    pltpu.sync_copy(zeros_h, zbuf)

    @pl.loop(0, N_FLUSH)
    def _(f):
        pltpu.sync_copy(zbuf, accum.at[pl.ds(nbase + f * FLUSH_CHUNK,
                                             FLUSH_CHUNK)])

    plsc.subcore_barrier()

    t0, t1 = tbl_pair

    def fire(chunk, buf):
        @pl.when(c == 0)
        def _():
            pltpu.async_copy(t0.at[idx_s.at[chunk]], buf, gsem)

        @pl.when(c == 1)
        def _():
            pltpu.async_copy(t1.at[idx_s.at[chunk]], buf, gsem)

    def wait_gather(buf):
        pltpu.make_async_copy(t0.at[idx_s.at[0]], buf, gsem).wait()

    def scat(chunk, buf):
        pltpu.sync_copy(buf, accum.at[idx_d.at[chunk]], add=True)

    fire(0, rowsA)

    @pl.loop(0, (CHUNKS_PER_TILE - 1) // 2)
    def _(j):
        a = 2 * j
        wait_gather(rowsA)
        fire(a + 1, rowsB)
        scat(a, rowsA)
        wait_gather(rowsB)
        fire(a + 2, rowsA)
        scat(a + 1, rowsB)

    wait_gather(rowsA)
    scat(CHUNKS_PER_TILE - 1, rowsA)

    plsc.subcore_barrier()

    o0, o1 = out_pair

    @pl.loop(0, N_FLUSH)
    def _(f):
        off = nbase + f * FLUSH_CHUNK
        pltpu.sync_copy(accum.at[pl.ds(off, FLUSH_CHUNK)], zbuf)

        @pl.when(c == 0)
        def _():
            pltpu.sync_copy(zbuf, o0.at[pl.ds(off, FLUSH_CHUNK)])

        @pl.when(c == 1)
        def _():
            pltpu.sync_copy(zbuf, o1.at[pl.ds(off, FLUSH_CHUNK)])

    plsc.subcore_barrier()


def _stage_indices(src_h, dst_h, s, idx_s, idx_d):
    pltpu.sync_copy(src_h.at[pl.ds(s * CHUNKS_PER_TILE, CHUNKS_PER_TILE)],
                    idx_s)
    pltpu.sync_copy(dst_h.at[pl.ds(s * CHUNKS_PER_TILE, CHUNKS_PER_TILE)],
                    idx_d)


def _seg_body_gc1(src_h, dst_h, zeros_h, t0, t1, out0, out1, *scr):
    c = lax.axis_index("c")
    s = lax.axis_index("s")
    _stage_indices(src_h, dst_h, s, scr[0], scr[1])
    _seg_pass(c, s, (t0, t1), (out0, out1), zeros_h, *scr)


def _seg_gc1(src2d, dst2d, zeros_h, t0, t1):
    mesh = plsc.VectorSubcoreMesh(core_axis_name="c", subcore_axis_name="s",
                                  num_cores=2, num_subcores=NUM_TILES)
    out = jax.ShapeDtypeStruct((N, 64), jnp.float32)
    f = pl.kernel(
        _seg_body_gc1,
        out_type=(out, out),
        mesh=mesh,
        compiler_params=pltpu.CompilerParams(use_tc_tiling_on_sc=False),
        scratch_types=_seg_scratch(),
    )
    return f(src2d, dst2d, zeros_h, t0, t1)


def _seg_body_gc2(src_h, dst_h, zeros_h, t0, t1, t2, t3,
                  out0, out1, out2, out3, *scr):
    c = lax.axis_index("c")
    s = lax.axis_index("s")
    _stage_indices(src_h, dst_h, s, scr[0], scr[1])
    _seg_pass(c, s, (t0, t2), (out0, out2), zeros_h, *scr)
    _seg_pass(c, s, (t1, t3), (out1, out3), zeros_h, *scr)


def _seg_gc2(src2d, dst2d, zeros_h, t0, t1, t2, t3):
    mesh = plsc.VectorSubcoreMesh(core_axis_name="c", subcore_axis_name="s",
                                  num_cores=2, num_subcores=NUM_TILES)
    out = jax.ShapeDtypeStruct((N, 64), jnp.float32)
    f = pl.kernel(
        _seg_body_gc2,
        out_type=(out, out, out, out),
        mesh=mesh,
        compiler_params=pltpu.CompilerParams(use_tc_tiling_on_sc=False),
        scratch_types=_seg_scratch(),
    )
    return f(src2d, dst2d, zeros_h, t0, t1, t2, t3)


ROWS_A = 1000


def _gc1_body(a0, a1, xb, wr, wx, b1, h1, g0, g1, g2, g3):
    agg = jnp.concatenate([a0[...], a1[...]], axis=1)
    h = jnp.dot(agg, wr[...], preferred_element_type=jnp.float32)
    h += jnp.dot(xb[...], wx[...], preferred_element_type=jnp.float32)
    h = jnp.maximum(h + b1[...], 0.0)
    h1[...] = h
    g0[...] = h[:, 0:64]
    g1[...] = h[:, 64:128]
    g2[...] = h[:, 128:192]
    g3[...] = h[:, 192:256]


def _gc1_dense(agg0, agg1, x, W_rel1, W_root1, b1):
    g_spec = pl.BlockSpec((ROWS_A, 64), lambda i: (i, 0))
    out64 = jax.ShapeDtypeStruct((N, 64), jnp.float32)
    return pl.pallas_call(
        _gc1_body,
        grid=(N // ROWS_A,),
        in_specs=[
            g_spec,
            g_spec,
            pl.BlockSpec((ROWS_A, IN), lambda i: (i, 0)),
            pl.BlockSpec((IN, HID), lambda i: (0, 0)),
            pl.BlockSpec((IN, HID), lambda i: (0, 0)),
            pl.BlockSpec((1, HID), lambda i: (0, 0)),
        ],
        out_specs=[pl.BlockSpec((ROWS_A, HID), lambda i: (i, 0)),
                   g_spec, g_spec, g_spec, g_spec],
        out_shape=[jax.ShapeDtypeStruct((N, HID), jnp.float32),
                   out64, out64, out64, out64],
    )(agg0, agg1, x, W_rel1, W_root1, b1)


def _gc2_body(a0, a1, a2, a3, h1b, wr, wx, b2, h2):
    agg = jnp.concatenate([a0[...], a1[...], a2[...], a3[...]], axis=1)
    h = jnp.dot(agg, wr[...], preferred_element_type=jnp.float32)
    h += jnp.dot(h1b[...], wx[...], preferred_element_type=jnp.float32)
    h2[...] = jnp.maximum(h + b2[...], 0.0)


def _gc2_dense(agg2s, h1, W_rel2, W_root2, b2):
    g_spec = pl.BlockSpec((ROWS_A, 64), lambda i: (i, 0))
    return pl.pallas_call(
        _gc2_body,
        grid=(N // ROWS_A,),
        in_specs=[
            g_spec, g_spec, g_spec, g_spec,
            pl.BlockSpec((ROWS_A, HID), lambda i: (i, 0)),
            pl.BlockSpec((HID, HID), lambda i: (0, 0)),
            pl.BlockSpec((HID, HID), lambda i: (0, 0)),
            pl.BlockSpec((1, HID), lambda i: (0, 0)),
        ],
        out_specs=pl.BlockSpec((ROWS_A, HID), lambda i: (i, 0)),
        out_shape=jax.ShapeDtypeStruct((N, HID), jnp.float32),
    )(*agg2s, h1, W_rel2, W_root2, b2)


K_HEAD = 12800
FLAT = N_PER * HID


def _heads_body(fb, wl, wls, bl, bls, loc_ref, ls_ref):
    i = pl.program_id(0)

    @pl.when(i == 0)
    def _():
        loc_ref[...] = jnp.broadcast_to(bl[...], (BATCH, LAT))
        ls_ref[...] = jnp.broadcast_to(bls[...], (BATCH, LAT))

    f = fb[...]
    loc_ref[...] += jnp.dot(f, wl[...], preferred_element_type=jnp.float32)
    ls_ref[...] += jnp.dot(f, wls[...], preferred_element_type=jnp.float32)


def _heads(flat, W_loc, b_loc, W_ls, b_ls):
    out_spec = pl.BlockSpec((BATCH, LAT), lambda i: (0, 0))
    out = jax.ShapeDtypeStruct((BATCH, LAT), jnp.float32)
    w_spec = pl.BlockSpec((K_HEAD, LAT), lambda i: (i, 0))
    return pl.pallas_call(
        _heads_body,
        grid=(FLAT // K_HEAD,),
        in_specs=[
            pl.BlockSpec((BATCH, K_HEAD), lambda i: (0, i)),
            w_spec, w_spec,
            pl.BlockSpec((1, LAT), lambda i: (0, 0)),
            pl.BlockSpec((1, LAT), lambda i: (0, 0)),
        ],
        out_specs=[out_spec, out_spec],
        out_shape=[out, out],
        compiler_params=pltpu.CompilerParams(
            dimension_semantics=("arbitrary",)),
    )(flat, W_loc, W_ls, b_loc, b_ls)


@jax.jit
def kernel(x, edge_index, batch, W_rel1, b1, W_root1, W_rel2, b2, W_root2,
           W_loc, b_loc, W_ls, b_ls):
    src2d = edge_index[0].reshape(E // EDGE_CHUNK, EDGE_CHUNK)
    dst2d = edge_index[1].reshape(E // EDGE_CHUNK, EDGE_CHUNK)
    zeros_h = jnp.zeros((FLUSH_CHUNK, 64), jnp.float32)
    x0 = x[:, 0:64]
    x1 = x[:, 64:128]

    agg1_0, agg1_1 = _seg_gc1(src2d, dst2d, zeros_h, x0, x1)
    h1, hg0, hg1, hg2, hg3 = _gc1_dense(
        agg1_0, agg1_1, x, W_rel1, W_root1, b1.reshape(1, HID))
    agg2s = _seg_gc2(src2d, dst2d, zeros_h, hg0, hg1, hg2, hg3)
    h2 = _gc2_dense(agg2s, h1, W_rel2, W_root2, b2.reshape(1, HID))
    flat = h2.reshape(BATCH, FLAT)
    loc, ls = _heads(flat, W_loc, b_loc.reshape(1, LAT),
                     W_ls, b_ls.reshape(1, LAT))
    return loc, ls


# speedup vs baseline: 6.1914x; 1.2319x over previous
import jax
import jax.numpy as jnp
from jax import lax
from jax.experimental import pallas as pl
from jax.experimental.pallas import tpu as pltpu
from jax.experimental.pallas import tpu_sc as plsc

N = 16000
E = 256000
IN = 128
HID = 256
LAT = 64
BATCH = 16
N_PER = 1000

NUM_TILES = 16
EDGE_CHUNK = 128
CHUNKS_PER_TILE = E // (NUM_TILES * EDGE_CHUNK)
NODES_PER_TILE = N // NUM_TILES
FLUSH_CHUNK = 125
N_FLUSH = NODES_PER_TILE // FLUSH_CHUNK


def _seg_scratch():
    return [
        pltpu.VMEM((CHUNKS_PER_TILE, EDGE_CHUNK), jnp.int32),
        pltpu.VMEM((CHUNKS_PER_TILE, EDGE_CHUNK), jnp.int32),
        pltpu.VMEM((EDGE_CHUNK, 64), jnp.float32),
        pltpu.VMEM((EDGE_CHUNK, 64), jnp.float32),
        pltpu.VMEM((EDGE_CHUNK, 64), jnp.float32),
        pltpu.VMEM((FLUSH_CHUNK, 64), jnp.float32),
        pltpu.VMEM_SHARED((N, 64), jnp.float32),
        pltpu.SemaphoreType.DMA,
        pltpu.SemaphoreType.DMA,
    ]


def _seg_pass(c, s, tbl_pair, out_pair, zeros_h,
              idx_s, idx_d, rowsA, rowsB, rowsC, zbuf, accum, gsem, ssem):
    nbase = s * NODES_PER_TILE
    pltpu.sync_copy(zeros_h, zbuf)

    @pl.loop(0, N_FLUSH)
    def _(f):
        pltpu.sync_copy(zbuf, accum.at[pl.ds(nbase + f * FLUSH_CHUNK,
                                             FLUSH_CHUNK)])

    plsc.subcore_barrier()

    t0, t1 = tbl_pair
    bufs = (rowsA, rowsB, rowsC)

    def gf(chunk, q):
        @pl.when(c == 0)
        def _():
            pltpu.async_copy(t0.at[idx_s.at[chunk]], bufs[q], gsem)

        @pl.when(c == 1)
        def _():
            pltpu.async_copy(t1.at[idx_s.at[chunk]], bufs[q], gsem)

    def gw(q):
        pltpu.make_async_copy(t0.at[idx_s.at[0]], bufs[q], gsem).wait()

    def sf(chunk, q):
        pltpu.async_copy(bufs[q], accum.at[idx_d.at[chunk]], ssem, add=True)

    def sw():
        pltpu.make_async_copy(bufs[0], accum.at[idx_d.at[0]], ssem).wait()

    gf(0, 0)
    gf(1, 1)
    gw(0); sf(0, 0); gf(2, 2)
    gw(1); sf(1, 1); sw(); gf(3, 0)
    gw(2); sf(2, 2); sw(); gf(4, 1)

    @pl.loop(0, (CHUNKS_PER_TILE - 5) // 3)
    def _(j):
        n = 3 + 3 * j
        gw(0); sf(n, 0); sw(); gf(n + 2, 2)
        gw(1); sf(n + 1, 1); sw(); gf(n + 3, 0)
        gw(2); sf(n + 2, 2); sw(); gf(n + 4, 1)

    gw(0); sf(CHUNKS_PER_TILE - 2, 0); sw()
    gw(1); sf(CHUNKS_PER_TILE - 1, 1); sw()
    sw()

    plsc.subcore_barrier()

    o0, o1 = out_pair

    @pl.loop(0, N_FLUSH)
    def _(f):
        off = nbase + f * FLUSH_CHUNK
        pltpu.sync_copy(accum.at[pl.ds(off, FLUSH_CHUNK)], zbuf)

        @pl.when(c == 0)
        def _():
            pltpu.sync_copy(zbuf, o0.at[pl.ds(off, FLUSH_CHUNK)])

        @pl.when(c == 1)
        def _():
            pltpu.sync_copy(zbuf, o1.at[pl.ds(off, FLUSH_CHUNK)])

    plsc.subcore_barrier()


def _stage_indices(src_h, dst_h, s, idx_s, idx_d):
    pltpu.sync_copy(src_h.at[pl.ds(s * CHUNKS_PER_TILE, CHUNKS_PER_TILE)],
                    idx_s)
    pltpu.sync_copy(dst_h.at[pl.ds(s * CHUNKS_PER_TILE, CHUNKS_PER_TILE)],
                    idx_d)


def _seg_body_gc1(src_h, dst_h, zeros_h, t0, t1, out0, out1, *scr):
    c = lax.axis_index("c")
    s = lax.axis_index("s")
    _stage_indices(src_h, dst_h, s, scr[0], scr[1])
    _seg_pass(c, s, (t0, t1), (out0, out1), zeros_h, *scr)


def _seg_gc1(src2d, dst2d, zeros_h, t0, t1):
    mesh = plsc.VectorSubcoreMesh(core_axis_name="c", subcore_axis_name="s",
                                  num_cores=2, num_subcores=NUM_TILES)
    out = jax.ShapeDtypeStruct((N, 64), jnp.float32)
    f = pl.kernel(
        _seg_body_gc1,
        out_type=(out, out),
        mesh=mesh,
        compiler_params=pltpu.CompilerParams(use_tc_tiling_on_sc=False),
        scratch_types=_seg_scratch(),
    )
    return f(src2d, dst2d, zeros_h, t0, t1)


def _seg_body_gc2(src_h, dst_h, zeros_h, t0, t1, t2, t3,
                  out0, out1, out2, out3, *scr):
    c = lax.axis_index("c")
    s = lax.axis_index("s")
    _stage_indices(src_h, dst_h, s, scr[0], scr[1])
    _seg_pass(c, s, (t0, t2), (out0, out2), zeros_h, *scr)
    _seg_pass(c, s, (t1, t3), (out1, out3), zeros_h, *scr)


def _seg_gc2(src2d, dst2d, zeros_h, t0, t1, t2, t3):
    mesh = plsc.VectorSubcoreMesh(core_axis_name="c", subcore_axis_name="s",
                                  num_cores=2, num_subcores=NUM_TILES)
    out = jax.ShapeDtypeStruct((N, 64), jnp.float32)
    f = pl.kernel(
        _seg_body_gc2,
        out_type=(out, out, out, out),
        mesh=mesh,
        compiler_params=pltpu.CompilerParams(use_tc_tiling_on_sc=False),
        scratch_types=_seg_scratch(),
    )
    return f(src2d, dst2d, zeros_h, t0, t1, t2, t3)


ROWS_A = 1000


def _gc1_body(a0, a1, xb, wr, wx, b1, h1, g0, g1, g2, g3):
    agg = jnp.concatenate([a0[...], a1[...]], axis=1)
    h = jnp.dot(agg, wr[...], preferred_element_type=jnp.float32)
    h += jnp.dot(xb[...], wx[...], preferred_element_type=jnp.float32)
    h = jnp.maximum(h + b1[...], 0.0)
    h1[...] = h
    g0[...] = h[:, 0:64]
    g1[...] = h[:, 64:128]
    g2[...] = h[:, 128:192]
    g3[...] = h[:, 192:256]


def _gc1_dense(agg0, agg1, x, W_rel1, W_root1, b1):
    g_spec = pl.BlockSpec((ROWS_A, 64), lambda i: (i, 0))
    out64 = jax.ShapeDtypeStruct((N, 64), jnp.float32)
    return pl.pallas_call(
        _gc1_body,
        grid=(N // ROWS_A,),
        in_specs=[
            g_spec,
            g_spec,
            pl.BlockSpec((ROWS_A, IN), lambda i: (i, 0)),
            pl.BlockSpec((IN, HID), lambda i: (0, 0)),
            pl.BlockSpec((IN, HID), lambda i: (0, 0)),
            pl.BlockSpec((1, HID), lambda i: (0, 0)),
        ],
        out_specs=[pl.BlockSpec((ROWS_A, HID), lambda i: (i, 0)),
                   g_spec, g_spec, g_spec, g_spec],
        out_shape=[jax.ShapeDtypeStruct((N, HID), jnp.float32),
                   out64, out64, out64, out64],
    )(agg0, agg1, x, W_rel1, W_root1, b1)


def _gc2_body(a0, a1, a2, a3, h1b, wr, wx, b2, h2):
    agg = jnp.concatenate([a0[...], a1[...], a2[...], a3[...]], axis=1)
    h = jnp.dot(agg, wr[...], preferred_element_type=jnp.float32)
    h += jnp.dot(h1b[...], wx[...], preferred_element_type=jnp.float32)
    h2[...] = jnp.maximum(h + b2[...], 0.0)


def _gc2_dense(agg2s, h1, W_rel2, W_root2, b2):
    g_spec = pl.BlockSpec((ROWS_A, 64), lambda i: (i, 0))
    return pl.pallas_call(
        _gc2_body,
        grid=(N // ROWS_A,),
        in_specs=[
            g_spec, g_spec, g_spec, g_spec,
            pl.BlockSpec((ROWS_A, HID), lambda i: (i, 0)),
            pl.BlockSpec((HID, HID), lambda i: (0, 0)),
            pl.BlockSpec((HID, HID), lambda i: (0, 0)),
            pl.BlockSpec((1, HID), lambda i: (0, 0)),
        ],
        out_specs=pl.BlockSpec((ROWS_A, HID), lambda i: (i, 0)),
        out_shape=jax.ShapeDtypeStruct((N, HID), jnp.float32),
    )(*agg2s, h1, W_rel2, W_root2, b2)


K_HEAD = 12800
FLAT = N_PER * HID


def _heads_body(fb, wl, wls, bl, bls, loc_ref, ls_ref):
    i = pl.program_id(0)

    @pl.when(i == 0)
    def _():
        loc_ref[...] = jnp.broadcast_to(bl[...], (BATCH, LAT))
        ls_ref[...] = jnp.broadcast_to(bls[...], (BATCH, LAT))

    f = fb[...]
    loc_ref[...] += jnp.dot(f, wl[...], preferred_element_type=jnp.float32)
    ls_ref[...] += jnp.dot(f, wls[...], preferred_element_type=jnp.float32)


def _heads(flat, W_loc, b_loc, W_ls, b_ls):
    out_spec = pl.BlockSpec((BATCH, LAT), lambda i: (0, 0))
    out = jax.ShapeDtypeStruct((BATCH, LAT), jnp.float32)
    w_spec = pl.BlockSpec((K_HEAD, LAT), lambda i: (i, 0))
    return pl.pallas_call(
        _heads_body,
        grid=(FLAT // K_HEAD,),
        in_specs=[
            pl.BlockSpec((BATCH, K_HEAD), lambda i: (0, i)),
            w_spec, w_spec,
            pl.BlockSpec((1, LAT), lambda i: (0, 0)),
            pl.BlockSpec((1, LAT), lambda i: (0, 0)),
        ],
        out_specs=[out_spec, out_spec],
        out_shape=[out, out],
        compiler_params=pltpu.CompilerParams(
            dimension_semantics=("arbitrary",)),
    )(flat, W_loc, W_ls, b_loc, b_ls)


@jax.jit
def kernel(x, edge_index, batch, W_rel1, b1, W_root1, W_rel2, b2, W_root2,
           W_loc, b_loc, W_ls, b_ls):
    src2d = edge_index[0].reshape(E // EDGE_CHUNK, EDGE_CHUNK)
    dst2d = edge_index[1].reshape(E // EDGE_CHUNK, EDGE_CHUNK)
    zeros_h = jnp.zeros((FLUSH_CHUNK, 64), jnp.float32)
    x0 = x[:, 0:64]
    x1 = x[:, 64:128]

    agg1_0, agg1_1 = _seg_gc1(src2d, dst2d, zeros_h, x0, x1)
    h1, hg0, hg1, hg2, hg3 = _gc1_dense(
        agg1_0, agg1_1, x, W_rel1, W_root1, b1.reshape(1, HID))
    agg2s = _seg_gc2(src2d, dst2d, zeros_h, hg0, hg1, hg2, hg3)
    h2 = _gc2_dense(agg2s, h1, W_rel2, W_root2, b2.reshape(1, HID))
    flat = h2.reshape(BATCH, FLAT)
    loc, ls = _heads(flat, W_loc, b_loc.reshape(1, LAT),
                     W_ls, b_ls.reshape(1, LAT))
    return loc, ls


# speedup vs baseline: 6.4589x; 1.0432x over previous
import jax
import jax.numpy as jnp
from jax import lax
from jax.experimental import pallas as pl
from jax.experimental.pallas import tpu as pltpu
from jax.experimental.pallas import tpu_sc as plsc

N = 16000
E = 256000
IN = 128
HID = 256
LAT = 64
BATCH = 16
N_PER = 1000

NUM_TILES = 16
EDGE_CHUNK = 128
CHUNKS_PER_TILE = E // (NUM_TILES * EDGE_CHUNK)
NODES_PER_TILE = N // NUM_TILES
FLUSH_CHUNK = 125
N_FLUSH = NODES_PER_TILE // FLUSH_CHUNK


def _seg_scratch():
    return [
        pltpu.VMEM((CHUNKS_PER_TILE, EDGE_CHUNK), jnp.int32),
        pltpu.VMEM((CHUNKS_PER_TILE, EDGE_CHUNK), jnp.int32),
        pltpu.VMEM((EDGE_CHUNK, 64), jnp.float32),
        pltpu.VMEM((EDGE_CHUNK, 64), jnp.float32),
        pltpu.VMEM((EDGE_CHUNK, 64), jnp.float32),
        pltpu.VMEM((FLUSH_CHUNK, 64), jnp.float32),
        pltpu.VMEM_SHARED((N, 64), jnp.float32),
        pltpu.SemaphoreType.DMA,
        pltpu.SemaphoreType.DMA,
    ]


def _seg_pass(c, s, tbl_pair, out_pair, zeros_h,
              idx_s, idx_d, rowsA, rowsB, rowsC, zbuf, accum, gsem, ssem):
    nbase = s * NODES_PER_TILE
    pltpu.sync_copy(zeros_h, zbuf)

    @pl.loop(0, N_FLUSH)
    def _(f):
        pltpu.sync_copy(zbuf, accum.at[pl.ds(nbase + f * FLUSH_CHUNK,
                                             FLUSH_CHUNK)])

    plsc.subcore_barrier()

    t0, t1 = tbl_pair
    bufs = (rowsA, rowsB, rowsC)

    def gf(chunk, q):
        @pl.when(c == 0)
        def _():
            pltpu.async_copy(t0.at[idx_s.at[chunk]], bufs[q], gsem)

        @pl.when(c == 1)
        def _():
            pltpu.async_copy(t1.at[idx_s.at[chunk]], bufs[q], gsem)

    def gw(q):
        pltpu.make_async_copy(t0.at[idx_s.at[0]], bufs[q], gsem).wait()

    def sf(chunk, q):
        pltpu.async_copy(bufs[q], accum.at[idx_d.at[chunk]], ssem, add=True)

    def sw():
        pltpu.make_async_copy(bufs[0], accum.at[idx_d.at[0]], ssem).wait()

    gf(0, 0)
    gf(1, 1)
    gw(0); sf(0, 0); gf(2, 2)
    gw(1); sf(1, 1); sw(); gf(3, 0)
    gw(2); sf(2, 2); sw(); gf(4, 1)

    @pl.loop(0, (CHUNKS_PER_TILE - 5) // 3)
    def _(j):
        n = 3 + 3 * j
        gw(0); sf(n, 0); sw(); gf(n + 2, 2)
        gw(1); sf(n + 1, 1); sw(); gf(n + 3, 0)
        gw(2); sf(n + 2, 2); sw(); gf(n + 4, 1)

    gw(0); sf(CHUNKS_PER_TILE - 2, 0); sw()
    gw(1); sf(CHUNKS_PER_TILE - 1, 1); sw()
    sw()

    plsc.subcore_barrier()

    o0, o1 = out_pair

    @pl.loop(0, N_FLUSH)
    def _(f):
        off = nbase + f * FLUSH_CHUNK
        pltpu.sync_copy(accum.at[pl.ds(off, FLUSH_CHUNK)], zbuf)

        @pl.when(c == 0)
        def _():
            pltpu.sync_copy(zbuf, o0.at[pl.ds(off, FLUSH_CHUNK)])

        @pl.when(c == 1)
        def _():
            pltpu.sync_copy(zbuf, o1.at[pl.ds(off, FLUSH_CHUNK)])

    plsc.subcore_barrier()


def _stage_indices(src_h, dst_h, s, idx_s, idx_d):
    pltpu.sync_copy(src_h.at[pl.ds(s * CHUNKS_PER_TILE, CHUNKS_PER_TILE)],
                    idx_s)
    pltpu.sync_copy(dst_h.at[pl.ds(s * CHUNKS_PER_TILE, CHUNKS_PER_TILE)],
                    idx_d)


def _seg_body_gc1(src_h, dst_h, zeros_h, t0, t1, out0, out1, *scr):
    c = lax.axis_index("c")
    s = lax.axis_index("s")
    _stage_indices(src_h, dst_h, s, scr[0], scr[1])
    _seg_pass(c, s, (t0, t1), (out0, out1), zeros_h, *scr)


def _seg_gc1(src2d, dst2d, zeros_h, t0, t1):
    mesh = plsc.VectorSubcoreMesh(core_axis_name="c", subcore_axis_name="s",
                                  num_cores=2, num_subcores=NUM_TILES)
    out = jax.ShapeDtypeStruct((N, 64), jnp.float32)
    f = pl.kernel(
        _seg_body_gc1,
        out_type=(out, out),
        mesh=mesh,
        compiler_params=pltpu.CompilerParams(use_tc_tiling_on_sc=False),
        scratch_types=_seg_scratch(),
    )
    return f(src2d, dst2d, zeros_h, t0, t1)


def _seg_body_gc2(src_h, dst_h, zeros_h, t0, t1, t2, t3,
                  out0, out1, out2, out3, *scr):
    c = lax.axis_index("c")
    s = lax.axis_index("s")
    _stage_indices(src_h, dst_h, s, scr[0], scr[1])
    _seg_pass(c, s, (t0, t2), (out0, out2), zeros_h, *scr)
    _seg_pass(c, s, (t1, t3), (out1, out3), zeros_h, *scr)


def _seg_gc2(src2d, dst2d, zeros_h, t0, t1, t2, t3):
    mesh = plsc.VectorSubcoreMesh(core_axis_name="c", subcore_axis_name="s",
                                  num_cores=2, num_subcores=NUM_TILES)
    out = jax.ShapeDtypeStruct((N, 64), jnp.float32)
    f = pl.kernel(
        _seg_body_gc2,
        out_type=(out, out, out, out),
        mesh=mesh,
        compiler_params=pltpu.CompilerParams(use_tc_tiling_on_sc=False),
        scratch_types=_seg_scratch(),
    )
    return f(src2d, dst2d, zeros_h, t0, t1, t2, t3)


ROWS_A = 1000


def _gc1_body(a0, a1, xb, wr, wx, b1, h1, g0, g1, g2, g3):
    agg = jnp.concatenate([a0[...], a1[...]], axis=1)
    h = jnp.dot(agg, wr[...], preferred_element_type=jnp.float32)
    h += jnp.dot(xb[...], wx[...], preferred_element_type=jnp.float32)
    h = jnp.maximum(h + b1[...], 0.0)
    h1[...] = h
    g0[...] = h[:, 0:64]
    g1[...] = h[:, 64:128]
    g2[...] = h[:, 128:192]
    g3[...] = h[:, 192:256]


def _gc1_dense(agg0, agg1, x, W_rel1, W_root1, b1):
    g_spec = pl.BlockSpec((ROWS_A, 64), lambda i: (i, 0))
    out64 = jax.ShapeDtypeStruct((N, 64), jnp.float32)
    return pl.pallas_call(
        _gc1_body,
        grid=(N // ROWS_A,),
        in_specs=[
            g_spec,
            g_spec,
            pl.BlockSpec((ROWS_A, IN), lambda i: (i, 0)),
            pl.BlockSpec((IN, HID), lambda i: (0, 0)),
            pl.BlockSpec((IN, HID), lambda i: (0, 0)),
            pl.BlockSpec((1, HID), lambda i: (0, 0)),
        ],
        out_specs=[pl.BlockSpec((ROWS_A, HID), lambda i: (i, 0)),
                   g_spec, g_spec, g_spec, g_spec],
        out_shape=[jax.ShapeDtypeStruct((N, HID), jnp.float32),
                   out64, out64, out64, out64],
    )(agg0, agg1, x, W_rel1, W_root1, b1)


NL_B = 40
FLAT = N_PER * HID


def _gc2_heads_body(a0, a1, a2, a3, h1b, wr, wx, b2, wl, wls, bl, bls,
                    loc_ref, ls_ref):
    i = pl.program_id(0)
    agg = jnp.concatenate([a0[...], a1[...], a2[...], a3[...]],
                          axis=2).reshape(BATCH * NL_B, HID)
    h = jnp.dot(agg, wr[...], preferred_element_type=jnp.float32)
    h += jnp.dot(h1b[...].reshape(BATCH * NL_B, HID), wx[...],
                 preferred_element_type=jnp.float32)
    h2 = jnp.maximum(h + b2[...], 0.0)
    flat = h2.reshape(BATCH, NL_B * HID)

    @pl.when(i == 0)
    def _():
        loc_ref[...] = jnp.broadcast_to(bl[...], (BATCH, LAT))
        ls_ref[...] = jnp.broadcast_to(bls[...], (BATCH, LAT))

    loc_ref[...] += jnp.dot(flat, wl[...].reshape(NL_B * HID, LAT),
                            preferred_element_type=jnp.float32)
    ls_ref[...] += jnp.dot(flat, wls[...].reshape(NL_B * HID, LAT),
                           preferred_element_type=jnp.float32)


def _gc2_heads(agg2s, h1, W_rel2, W_root2, b2, W_loc, b_loc, W_ls, b_ls):
    a_spec = pl.BlockSpec((BATCH, NL_B, 64), lambda i: (0, i, 0))
    w_spec = pl.BlockSpec((NL_B, HID, LAT), lambda i: (i, 0, 0))
    out_spec = pl.BlockSpec((BATCH, LAT), lambda i: (0, 0))
    out = jax.ShapeDtypeStruct((BATCH, LAT), jnp.float32)
    a3d = [a.reshape(BATCH, N_PER, 64) for a in agg2s]
    return pl.pallas_call(
        _gc2_heads_body,
        grid=(N_PER // NL_B,),
        in_specs=[
            a_spec, a_spec, a_spec, a_spec,
            pl.BlockSpec((BATCH, NL_B, HID), lambda i: (0, i, 0)),
            pl.BlockSpec((HID, HID), lambda i: (0, 0)),
            pl.BlockSpec((HID, HID), lambda i: (0, 0)),
            pl.BlockSpec((1, HID), lambda i: (0, 0)),
            w_spec, w_spec,
            pl.BlockSpec((1, LAT), lambda i: (0, 0)),
            pl.BlockSpec((1, LAT), lambda i: (0, 0)),
        ],
        out_specs=[out_spec, out_spec],
        out_shape=[out, out],
        compiler_params=pltpu.CompilerParams(
            dimension_semantics=("arbitrary",)),
    )(*a3d, h1.reshape(BATCH, N_PER, HID), W_rel2, W_root2, b2,
      W_loc.reshape(N_PER, HID, LAT), W_ls.reshape(N_PER, HID, LAT),
      b_loc, b_ls)


@jax.jit
def kernel(x, edge_index, batch, W_rel1, b1, W_root1, W_rel2, b2, W_root2,
           W_loc, b_loc, W_ls, b_ls):
    src2d = edge_index[0].reshape(E // EDGE_CHUNK, EDGE_CHUNK)
    dst2d = edge_index[1].reshape(E // EDGE_CHUNK, EDGE_CHUNK)
    zeros_h = jnp.zeros((FLUSH_CHUNK, 64), jnp.float32)
    x0 = x[:, 0:64]
    x1 = x[:, 64:128]

    agg1_0, agg1_1 = _seg_gc1(src2d, dst2d, zeros_h, x0, x1)
    h1, hg0, hg1, hg2, hg3 = _gc1_dense(
        agg1_0, agg1_1, x, W_rel1, W_root1, b1.reshape(1, HID))
    agg2s = _seg_gc2(src2d, dst2d, zeros_h, hg0, hg1, hg2, hg3)
    loc, ls = _gc2_heads(agg2s, h1, W_rel2, W_root2, b2.reshape(1, HID),
                         W_loc, b_loc.reshape(1, LAT), W_ls,
                         b_ls.reshape(1, LAT))
    return loc, ls
